# baseline (speedup 1.0000x reference)
import functools

import numpy as np
import jax
import jax.numpy as jnp
from jax.experimental import pallas as pl

_RATIO_SCALE = [(1.0 / 3, 1), (0.5, 1), (1, 1), (1, 1.5), (2, 1), (3, 1)]
_LEVELS = [(128, 128, 8.0), (64, 64, 16.0), (32, 32, 32.0), (16, 16, 64.0)]
_SIZES = [32, 64, 128, 256]
_NUM_ROWS = sum(h * w * 6 for (h, w, _) in _LEVELS)
_OFFS = [0, 98304, 122880, 129024]
_NBLK = 10
_BLK = _NUM_ROWS // _NBLK


def _boxes(level: int) -> np.ndarray:
    anch = np.zeros((6, 2), dtype=np.float32)
    for i, (ratio, scale) in enumerate(_RATIO_SCALE):
        anch[i, 0] = scale * _SIZES[level] * np.sqrt(ratio)
        anch[i, 1] = scale * _SIZES[level] / np.sqrt(ratio)
    return anch


def _body(out_ref):
    i = pl.program_id(0)
    m = jax.lax.broadcasted_iota(jnp.int32, (4, _BLK), 1) + i * _BLK
    c = jax.lax.broadcasted_iota(jnp.int32, (4, _BLK), 0)
    ge1, ge2, ge3 = (m >= _OFFS[1]), (m >= _OFFS[2]), (m >= _OFFS[3])

    def per_level(v0, v1, v2, v3, dt):
        return jnp.where(ge3, dt(v3),
                         jnp.where(ge2, dt(v2), jnp.where(ge1, dt(v1),
                                                          dt(v0))))

    i32, f32 = jnp.int32, jnp.float32
    n = m - per_level(*_OFFS, i32)
    n6f = jnp.floor(n.astype(f32) * f32(1.0 / 6.0))
    n6 = n6f.astype(i32)
    a = n - 6 * n6
    s = per_level(8.0, 16.0, 32.0, 64.0, f32)
    xf = (n6 & per_level(127, 63, 31, 15, i32)).astype(f32)
    yf = (n6 >> per_level(7, 6, 5, 4, i32)).astype(f32)
    cx = (xf + f32(0.5)) * s
    cy = (yf + f32(0.5)) * s
    b = _boxes(0)
    c_is_w = c == 2
    boxv = jnp.where(c_is_w, f32(b[0, 0]), f32(b[0, 1]))
    for k in range(1, 6):
        bc_k = jnp.where(c_is_w, f32(b[k, 0]), f32(b[k, 1]))
        boxv = jnp.where(a == k, bc_k, boxv)
    boxv = boxv * (s * f32(0.125))
    out_ref[...] = jnp.where(
        c == 0, cx, jnp.where(c == 1, cy, boxv))


@functools.cache
def _call():
    return pl.pallas_call(
        _body,
        out_shape=jax.ShapeDtypeStruct((4, _NUM_ROWS), jnp.float32),
        out_specs=pl.BlockSpec((4, _BLK), lambda i: (0, i)),
        grid=(_NBLK,),
    )


def kernel(feat0, feat1, feat2, feat3, x):
    del feat0, feat1, feat2, feat3, x
    return _call()().T


# speedup vs baseline: 1.0005x; 1.0005x over previous
import functools

import numpy as np
import jax
import jax.numpy as jnp
from jax.experimental import pallas as pl

_RATIO_SCALE = [(1.0 / 3, 1), (0.5, 1), (1, 1), (1, 1.5), (2, 1), (3, 1)]
_LEVELS = [(128, 128, 8.0), (64, 64, 16.0), (32, 32, 32.0), (16, 16, 64.0)]
_SIZES = [32, 64, 128, 256]
_NUM_ROWS = sum(h * w * 6 for (h, w, _) in _LEVELS)
_OFFS = [0, 98304, 122880, 129024]
_NBLK = 10
_BLK = _NUM_ROWS // _NBLK


def _boxes(level: int) -> np.ndarray:
    anch = np.zeros((6, 2), dtype=np.float32)
    for i, (ratio, scale) in enumerate(_RATIO_SCALE):
        anch[i, 0] = scale * _SIZES[level] * np.sqrt(ratio)
        anch[i, 1] = scale * _SIZES[level] / np.sqrt(ratio)
    return anch


def _body(out_ref):
    i = pl.program_id(0)
    m = jax.lax.broadcasted_iota(jnp.int32, (4, _BLK), 1) + i * _BLK
    c = jax.lax.broadcasted_iota(jnp.int32, (4, _BLK), 0)
    ge1, ge2, ge3 = (m >= _OFFS[1]), (m >= _OFFS[2]), (m >= _OFFS[3])

    def per_level(v0, v1, v2, v3, dt):
        return jnp.where(ge3, dt(v3),
                         jnp.where(ge2, dt(v2), jnp.where(ge1, dt(v1),
                                                          dt(v0))))

    i32, f32 = jnp.int32, jnp.float32
    n = m - per_level(*_OFFS, i32)
    n6f = jnp.floor(n.astype(f32) * f32(1.0 / 6.0))
    n6 = n6f.astype(i32)
    a = n - 6 * n6
    s = per_level(8.0, 16.0, 32.0, 64.0, f32)
    xf = (n6 & per_level(127, 63, 31, 15, i32)).astype(f32)
    yf = (n6 >> per_level(7, 6, 5, 4, i32)).astype(f32)
    cx = (xf + f32(0.5)) * s
    cy = (yf + f32(0.5)) * s
    b = _boxes(0)
    c1 = jax.lax.broadcasted_iota(jnp.int32, (4, 1), 0)
    c_is_w = c1 == 2
    boxv = jnp.where(c_is_w, f32(b[0, 0]), f32(b[0, 1]))
    for k in range(1, 6):
        bc_k = jnp.where(c_is_w, f32(b[k, 0]), f32(b[k, 1]))
        boxv = jnp.where(a == k, bc_k, boxv)
    boxv = boxv * (s * f32(0.125))
    out_ref[...] = jnp.where(
        c1 == 0, cx, jnp.where(c1 == 1, cy, boxv))


@functools.cache
def _call():
    return pl.pallas_call(
        _body,
        out_shape=jax.ShapeDtypeStruct((4, _NUM_ROWS), jnp.float32),
        out_specs=pl.BlockSpec((4, _BLK), lambda i: (0, i)),
        grid=(_NBLK,),
    )


def kernel(feat0, feat1, feat2, feat3, x):
    del feat0, feat1, feat2, feat3, x
    return _call()().T
